# baseline (speedup 1.0000x reference)
import functools

import jax
import jax.numpy as jnp
from jax import lax
from jax.experimental import pallas as pl
from jax.experimental.pallas import tpu as pltpu
from jax.experimental.pallas import tpu_sc as plsc

_EPS = 1e-06
_NSEG = 16
_LN2 = 0.6931471805599453
_C1 = 1.4415923923106588
_C2 = -0.7072548989690162
_C3 = 0.4115641479248821
_C4 = -0.18983442828200595
_C5 = 0.04392909981021807

_NW = 32
_BLK = 256


def _pow_p(v, pln2):
    v = jnp.maximum(v, _EPS)
    iv = lax.bitcast_convert_type(v, jnp.int32)
    e = (iv >> 23) - 127
    m = lax.bitcast_convert_type((iv & 0x007FFFFF) | 0x3F800000, jnp.float32)
    t = m - 1.0
    lg = t * (_C1 + t * (_C2 + t * (_C3 + t * (_C4 + t * _C5))))
    lg = lg + e.astype(jnp.float32)
    return jnp.exp(pln2 * lg)


@functools.lru_cache(maxsize=None)
def _make_sc_pool(total, d, t0=0):
    rows = (total - t0) // _NW
    nblk = rows // _BLK
    nch = d // 16
    nidc = rows // 16
    mesh = plsc.VectorSubcoreMesh(core_axis_name="c", subcore_axis_name="s")

    @functools.partial(
        pl.kernel,
        mesh=mesh,
        out_type=[
            jax.ShapeDtypeStruct((_NW, _NSEG, d), jnp.float32),
            jax.ShapeDtypeStruct((_NW, _NSEG), jnp.float32),
        ],
        scratch_types=[
            pltpu.VMEM((_BLK, d), jnp.float32),
            pltpu.VMEM((rows + 16,), jnp.int32),
            pltpu.VMEM((_NSEG, d), jnp.float32),
            pltpu.VMEM((16,), jnp.float32),
            pltpu.VMEM((48,), jnp.int32),
            pltpu.VMEM((16,), jnp.float32),
        ],
    )
    def sc_pool(x_hbm, bid_hbm, pln2_hbm, psum_hbm, pcnt_hbm,
                xbuf, bidv, acc, cntf, sbuf, pv):
        wid = lax.axis_index("s") * 2 + lax.axis_index("c")
        base = t0 + wid * rows
        pltpu.sync_copy(bid_hbm.at[pl.ds(base, rows)], bidv.at[pl.ds(0, rows)])
        pltpu.sync_copy(pln2_hbm, pv)
        pln2 = pv[...]
        zero16 = jnp.zeros((16,), jnp.float32)

        def searchsorted(sval):
            lo = jnp.int32(0)
            hi = jnp.int32(rows)
            for _ in range(11):
                mid = jnp.minimum((lo + hi) >> 1, rows - 1)
                v = bidv[pl.ds(mid, 16)][0]
                upd = hi > lo
                lt = v < sval
                lo = jnp.where(upd & lt, mid + 1, lo)
                hi = jnp.where(upd & (~lt), mid, hi)
            return lo

        bounds = [jnp.int32(0)] + [searchsorted(s) for s in range(1, 16)]
        seg_iota = lax.iota(jnp.int32, 16)
        starts = jnp.zeros((16,), jnp.int32)
        ends = jnp.full((16,), rows, jnp.int32)
        for s in range(1, 16):
            bs = jnp.full((16,), bounds[s], jnp.int32)
            starts = jnp.where(seg_iota == s, bs, starts)
            ends = jnp.where(seg_iota == s - 1, bs, ends)
        sbuf[pl.ds(0, 16)] = starts
        sbuf[pl.ds(16, 16)] = ends
        sbuf[pl.ds(32, 16)] = jnp.zeros((16,), jnp.int32)
        cntf[...] = (ends - starts).astype(jnp.float32)

        def zacc_body(s, carry):
            for j in range(nch):
                acc[s, pl.ds(j * 16, 16)] = zero16
            return carry

        lax.fori_loop(0, _NSEG, zacc_body, 0)

        def block_body(b, carry):
            pltpu.sync_copy(x_hbm.at[pl.ds(base + b * _BLK, _BLK)], xbuf)
            blo = b * _BLK

            def seg_body(s, carry2):
                st = sbuf[pl.ds(s, 16)][0]
                en = sbuf[pl.ds(16 + s, 16)][0]
                lo = jnp.clip(st, blo, blo + _BLK) - blo
                hi = jnp.clip(en, blo, blo + _BLK) - blo

                def row_body(r, accs):
                    return tuple(
                        accs[j] + _pow_p(xbuf[r, pl.ds(j * 16, 16)], pln2)
                        for j in range(nch)
                    )

                accs = lax.fori_loop(lo, hi, row_body,
                                     tuple(zero16 for _ in range(nch)))
                for j in range(nch):
                    acc[s, pl.ds(j * 16, 16)] += accs[j]
                return carry2

            lax.fori_loop(0, _NSEG, seg_body, 0)
            return carry

        lax.fori_loop(0, nblk, block_body, 0)
        pltpu.sync_copy(acc, psum_hbm.at[wid])
        pltpu.sync_copy(cntf, pcnt_hbm.at[wid])

    return sc_pool


@functools.lru_cache(maxsize=None)
def _make_sc_cube(total, d, t0=0):
    rows = (total - t0) // _NW
    blk = 128
    nblk = rows // blk
    nch = d // 16
    mesh = plsc.VectorSubcoreMesh(core_axis_name="c", subcore_axis_name="s")

    @functools.partial(
        pl.kernel,
        mesh=mesh,
        out_type=[
            jax.ShapeDtypeStruct((_NW, _NSEG, d), jnp.float32),
            jax.ShapeDtypeStruct((_NW, _NSEG), jnp.float32),
        ],
        scratch_types=[
            pltpu.VMEM((blk, d), jnp.float32),
            pltpu.VMEM((blk, d), jnp.float32),
            pltpu.VMEM((rows + 16,), jnp.int32),
            pltpu.VMEM((_NSEG, d), jnp.float32),
            pltpu.VMEM((16,), jnp.float32),
            pltpu.VMEM((48,), jnp.int32),
            pltpu.SemaphoreType.DMA,
            pltpu.SemaphoreType.DMA,
        ],
    )
    def sc_cube(x_hbm, bid_hbm, psum_hbm, pcnt_hbm,
                xbuf0, xbuf1, bidv, acc, cntf, sbuf, sem0, sem1):
        wid = lax.axis_index("s") * 2 + lax.axis_index("c")
        base = t0 + wid * rows
        bufs = (xbuf0, xbuf1)
        sems = (sem0, sem1)
        pltpu.async_copy(x_hbm.at[pl.ds(base, blk)], bufs[0], sems[0])
        pltpu.async_copy(x_hbm.at[pl.ds(base + blk, blk)], bufs[1], sems[1])
        pltpu.sync_copy(bid_hbm.at[pl.ds(base, rows)], bidv.at[pl.ds(0, rows)])
        zero16 = jnp.zeros((16,), jnp.float32)

        nsteps = max(1, (rows).bit_length())
        los = [jnp.int32(0)] * 15
        his = [jnp.int32(rows)] * 15
        for _ in range(nsteps):
            for k in range(15):
                lo, hi = los[k], his[k]
                mid = jnp.minimum((lo + hi) >> 1, rows - 1)
                v = bidv[pl.ds(mid, 16)][0]
                upd = hi > lo
                lt = v < (k + 1)
                los[k] = jnp.where(upd & lt, mid + 1, lo)
                his[k] = jnp.where(upd & (~lt), mid, hi)
        seg_iota = lax.iota(jnp.int32, 16)
        starts = jnp.zeros((16,), jnp.int32)
        ends = jnp.full((16,), rows, jnp.int32)
        for s in range(1, 16):
            bs = jnp.full((16,), los[s - 1], jnp.int32)
            starts = jnp.where(seg_iota == s, bs, starts)
            ends = jnp.where(seg_iota == s - 1, bs, ends)
        sbuf[pl.ds(0, 16)] = starts
        sbuf[pl.ds(16, 16)] = ends
        sbuf[pl.ds(32, 16)] = jnp.zeros((16,), jnp.int32)
        cntf[...] = (ends - starts).astype(jnp.float32)

        def zacc_body(s, carry):
            for j in range(nch):
                acc[s, pl.ds(j * 16, 16)] = zero16
            return carry

        lax.fori_loop(0, _NSEG, zacc_body, 0)

        def cube_row(xb, r, accs):
            out = []
            for j in range(nch):
                v = jnp.maximum(xb[r, pl.ds(j * 16, 16)], _EPS)
                out.append(accs[j] + v * v * v)
            return tuple(out)

        for b in range(nblk):
            xbuf = bufs[b % 2]
            pltpu.make_async_copy(x_hbm.at[pl.ds(0, blk)], xbuf,
                                  sems[b % 2]).wait()
            blo = b * blk

            def seg_body(s, carry2, _xbuf=xbuf, _blo=blo):
                st = sbuf[pl.ds(s, 16)][0]
                en = sbuf[pl.ds(16 + s, 16)][0]
                lo = jnp.clip(st, _blo, _blo + blk) - _blo
                hi = jnp.clip(en, _blo, _blo + blk) - _blo

                @pl.when(hi > lo)
                def _():
                    half = (hi - lo) >> 1

                    def pair_body(k, accs):
                        r0 = lo + 2 * k
                        return cube_row(_xbuf, r0 + 1,
                                        cube_row(_xbuf, r0, accs))

                    accs = lax.fori_loop(0, half, pair_body,
                                         tuple(zero16 for _ in range(nch)))
                    odd = ((hi - lo) & 1) == 1
                    for j in range(nch):
                        v = jnp.maximum(_xbuf[hi - 1, pl.ds(j * 16, 16)],
                                        _EPS)
                        tailv = jnp.where(odd, v * v * v, 0.0)
                        acc[s, pl.ds(j * 16, 16)] += accs[j] + tailv

                return carry2

            lax.fori_loop(0, _NSEG, seg_body, 0)

            if b + 2 < nblk:
                pltpu.async_copy(
                    x_hbm.at[pl.ds(base + (b + 2) * blk, blk)],
                    bufs[b % 2], sems[b % 2])
        pltpu.sync_copy(acc, psum_hbm.at[wid])
        pltpu.sync_copy(cntf, pcnt_hbm.at[wid])

    return sc_cube


_TCBLK = 2048


@functools.lru_cache(maxsize=None)
def _make_tc_partial(t0, d, cube):
    nblk = t0 // _TCBLK

    def body(x_ref, ids_ref, p_ref, psum_ref, pcnt_ref):
        @pl.when(pl.program_id(0) == 0)
        def _():
            psum_ref[...] = jnp.zeros_like(psum_ref)
            pcnt_ref[...] = jnp.zeros_like(pcnt_ref)

        ids = ids_ref[0, 0, :]
        smin = jnp.min(ids)
        smax = jnp.max(ids)
        v = jnp.maximum(x_ref[...], _EPS)
        feats = v * v * v if cube else v ** p_ref[0, 0]

        @pl.when(smin == smax)
        def _():
            colsum = jnp.sum(feats, axis=0)
            psum_ref[pl.ds(smin, 1), :] += colsum[None, :]
            pcnt_ref[pl.ds(smin, 1), :] += jnp.float32(_TCBLK)

        @pl.when(smin != smax)
        def _():
            oh = (ids[None, :] == lax.broadcasted_iota(
                jnp.int32, (_NSEG, _TCBLK), 0)).astype(jnp.float32)
            psum_ref[...] += jnp.dot(oh, feats,
                                     preferred_element_type=jnp.float32)
            pcnt_ref[...] += jnp.broadcast_to(
                jnp.sum(oh, axis=1)[:, None], (_NSEG, 128))

    return pl.pallas_call(
        body,
        grid=(nblk,),
        in_specs=[
            pl.BlockSpec((_TCBLK, d), lambda i: (i, 0)),
            pl.BlockSpec((1, 1, _TCBLK), lambda i: (i, 0, 0)),
            pl.BlockSpec((1, 1), lambda i: (0, 0)),
        ],
        out_specs=[
            pl.BlockSpec((_NSEG, d), lambda i: (0, 0)),
            pl.BlockSpec((_NSEG, 128), lambda i: (0, 0)),
        ],
        out_shape=[
            jax.ShapeDtypeStruct((_NSEG, d), jnp.float32),
            jax.ShapeDtypeStruct((_NSEG, 128), jnp.float32),
        ],
    )


def _finish_body(psum_ref, pcnt_ref, tsum_ref, tcnt_ref, p_ref, o_ref):
    p = p_ref[0, 0]
    s = jnp.sum(psum_ref[...], axis=0) + tsum_ref[...]
    c = (jnp.sum(pcnt_ref[...], axis=0) + tcnt_ref[:, 0])[:, None]
    mean = s / jnp.maximum(c, 1.0)
    o_ref[...] = mean ** (1.0 / p)


@functools.lru_cache(maxsize=None)
def _make_finish(d):
    return pl.pallas_call(
        _finish_body,
        out_shape=jax.ShapeDtypeStruct((_NSEG, d), jnp.float32),
    )


_T0 = 16384


def kernel(x, p, batch_ids):
    total, d = x.shape
    bid = batch_ids.astype(jnp.int32)
    bid2d = bid.reshape(total // _TCBLK, 1, _TCBLK)
    pf = p.astype(jnp.float32)
    p11 = pf.reshape(1, 1)
    pln2 = jnp.broadcast_to(pf * _LN2, (16,))

    def cube_branch(xx, bb, bb2, pp):
        ps, pc = _make_sc_cube(total, d, _T0)(xx, bb)
        ts, tc = _make_tc_partial(_T0, d, True)(xx, bb2, pp)
        return ps, pc, ts, tc

    def gen_branch(xx, bb, bb2, pp):
        ps, pc = _make_sc_pool(total, d, _T0)(xx, bb, pln2)
        ts, tc = _make_tc_partial(_T0, d, False)(xx, bb2, pp)
        return ps, pc, ts, tc

    psum, pcnt, tsum, tcnt = lax.cond(
        pf[0] == 3.0, cube_branch, gen_branch, x, bid, bid2d, p11)
    return _make_finish(d)(psum, pcnt, tsum, tcnt, p11)


# speedup vs baseline: 1.0458x; 1.0458x over previous
import functools

import jax
import jax.numpy as jnp
from jax import lax
from jax.experimental import pallas as pl
from jax.experimental.pallas import tpu as pltpu
from jax.experimental.pallas import tpu_sc as plsc

_EPS = 1e-06
_NSEG = 16
_LN2 = 0.6931471805599453
_C1 = 1.4415923923106588
_C2 = -0.7072548989690162
_C3 = 0.4115641479248821
_C4 = -0.18983442828200595
_C5 = 0.04392909981021807

_NW = 32
_BLK = 256


def _pow_p(v, pln2):
    v = jnp.maximum(v, _EPS)
    iv = lax.bitcast_convert_type(v, jnp.int32)
    e = (iv >> 23) - 127
    m = lax.bitcast_convert_type((iv & 0x007FFFFF) | 0x3F800000, jnp.float32)
    t = m - 1.0
    lg = t * (_C1 + t * (_C2 + t * (_C3 + t * (_C4 + t * _C5))))
    lg = lg + e.astype(jnp.float32)
    return jnp.exp(pln2 * lg)


@functools.lru_cache(maxsize=None)
def _make_sc_pool(total, d, t0=0):
    rows = (total - t0) // _NW
    nblk = rows // _BLK
    nch = d // 16
    nidc = rows // 16
    mesh = plsc.VectorSubcoreMesh(core_axis_name="c", subcore_axis_name="s")

    @functools.partial(
        pl.kernel,
        mesh=mesh,
        out_type=[
            jax.ShapeDtypeStruct((_NW, _NSEG, d), jnp.float32),
            jax.ShapeDtypeStruct((_NW, _NSEG), jnp.float32),
        ],
        scratch_types=[
            pltpu.VMEM((_BLK, d), jnp.float32),
            pltpu.VMEM((rows + 16,), jnp.int32),
            pltpu.VMEM((_NSEG, d), jnp.float32),
            pltpu.VMEM((16,), jnp.float32),
            pltpu.VMEM((48,), jnp.int32),
            pltpu.VMEM((16,), jnp.float32),
        ],
    )
    def sc_pool(x_hbm, bid_hbm, pln2_hbm, psum_hbm, pcnt_hbm,
                xbuf, bidv, acc, cntf, sbuf, pv):
        wid = lax.axis_index("s") * 2 + lax.axis_index("c")
        base = t0 + wid * rows
        pltpu.sync_copy(bid_hbm.at[pl.ds(base, rows)], bidv.at[pl.ds(0, rows)])
        pltpu.sync_copy(pln2_hbm, pv)
        pln2 = pv[...]
        zero16 = jnp.zeros((16,), jnp.float32)

        def searchsorted(sval):
            lo = jnp.int32(0)
            hi = jnp.int32(rows)
            for _ in range(11):
                mid = jnp.minimum((lo + hi) >> 1, rows - 1)
                v = bidv[pl.ds(mid, 16)][0]
                upd = hi > lo
                lt = v < sval
                lo = jnp.where(upd & lt, mid + 1, lo)
                hi = jnp.where(upd & (~lt), mid, hi)
            return lo

        bounds = [jnp.int32(0)] + [searchsorted(s) for s in range(1, 16)]
        seg_iota = lax.iota(jnp.int32, 16)
        starts = jnp.zeros((16,), jnp.int32)
        ends = jnp.full((16,), rows, jnp.int32)
        for s in range(1, 16):
            bs = jnp.full((16,), bounds[s], jnp.int32)
            starts = jnp.where(seg_iota == s, bs, starts)
            ends = jnp.where(seg_iota == s - 1, bs, ends)
        sbuf[pl.ds(0, 16)] = starts
        sbuf[pl.ds(16, 16)] = ends
        sbuf[pl.ds(32, 16)] = jnp.zeros((16,), jnp.int32)
        cntf[...] = (ends - starts).astype(jnp.float32)

        def zacc_body(s, carry):
            for j in range(nch):
                acc[s, pl.ds(j * 16, 16)] = zero16
            return carry

        lax.fori_loop(0, _NSEG, zacc_body, 0)

        def block_body(b, carry):
            pltpu.sync_copy(x_hbm.at[pl.ds(base + b * _BLK, _BLK)], xbuf)
            blo = b * _BLK

            def seg_body(s, carry2):
                st = sbuf[pl.ds(s, 16)][0]
                en = sbuf[pl.ds(16 + s, 16)][0]
                lo = jnp.clip(st, blo, blo + _BLK) - blo
                hi = jnp.clip(en, blo, blo + _BLK) - blo

                def row_body(r, accs):
                    return tuple(
                        accs[j] + _pow_p(xbuf[r, pl.ds(j * 16, 16)], pln2)
                        for j in range(nch)
                    )

                accs = lax.fori_loop(lo, hi, row_body,
                                     tuple(zero16 for _ in range(nch)))
                for j in range(nch):
                    acc[s, pl.ds(j * 16, 16)] += accs[j]
                return carry2

            lax.fori_loop(0, _NSEG, seg_body, 0)
            return carry

        lax.fori_loop(0, nblk, block_body, 0)
        pltpu.sync_copy(acc, psum_hbm.at[wid])
        pltpu.sync_copy(cntf, pcnt_hbm.at[wid])

    return sc_pool


@functools.lru_cache(maxsize=None)
def _make_sc_cube(total, d, t0=0):
    rows = (total - t0) // _NW
    blk = 128
    nblk = rows // blk
    nch = d // 16
    mesh = plsc.VectorSubcoreMesh(core_axis_name="c", subcore_axis_name="s")

    @functools.partial(
        pl.kernel,
        mesh=mesh,
        out_type=[
            jax.ShapeDtypeStruct((_NW, _NSEG, d), jnp.float32),
            jax.ShapeDtypeStruct((_NW, _NSEG), jnp.float32),
        ],
        scratch_types=[
            pltpu.VMEM((blk, d), jnp.float32),
            pltpu.VMEM((blk, d), jnp.float32),
            pltpu.VMEM((rows + 16,), jnp.int32),
            pltpu.VMEM((_NSEG, d), jnp.float32),
            pltpu.VMEM((16,), jnp.float32),
            pltpu.VMEM((48,), jnp.int32),
            pltpu.SemaphoreType.DMA,
            pltpu.SemaphoreType.DMA,
        ],
    )
    def sc_cube(x_hbm, bid_hbm, psum_hbm, pcnt_hbm,
                xbuf0, xbuf1, bidv, acc, cntf, sbuf, sem0, sem1):
        wid = lax.axis_index("s") * 2 + lax.axis_index("c")
        base = t0 + wid * rows
        bufs = (xbuf0, xbuf1)
        sems = (sem0, sem1)
        pltpu.async_copy(x_hbm.at[pl.ds(base, blk)], bufs[0], sems[0])
        pltpu.async_copy(x_hbm.at[pl.ds(base + blk, blk)], bufs[1], sems[1])
        pltpu.sync_copy(bid_hbm.at[pl.ds(base, rows)], bidv.at[pl.ds(0, rows)])
        zero16 = jnp.zeros((16,), jnp.float32)

        nsteps = max(1, (rows).bit_length())
        los = [jnp.int32(0)] * 15
        his = [jnp.int32(rows)] * 15
        for _ in range(nsteps):
            for k in range(15):
                lo, hi = los[k], his[k]
                mid = jnp.minimum((lo + hi) >> 1, rows - 1)
                v = bidv[pl.ds(mid, 16)][0]
                upd = hi > lo
                lt = v < (k + 1)
                los[k] = jnp.where(upd & lt, mid + 1, lo)
                his[k] = jnp.where(upd & (~lt), mid, hi)
        seg_iota = lax.iota(jnp.int32, 16)
        starts = jnp.zeros((16,), jnp.int32)
        ends = jnp.full((16,), rows, jnp.int32)
        for s in range(1, 16):
            bs = jnp.full((16,), los[s - 1], jnp.int32)
            starts = jnp.where(seg_iota == s, bs, starts)
            ends = jnp.where(seg_iota == s - 1, bs, ends)
        sbuf[pl.ds(0, 16)] = starts
        sbuf[pl.ds(16, 16)] = ends
        sbuf[pl.ds(32, 16)] = jnp.zeros((16,), jnp.int32)
        cntf[...] = (ends - starts).astype(jnp.float32)

        def zacc_body(s, carry):
            for j in range(nch):
                acc[s, pl.ds(j * 16, 16)] = zero16
            return carry

        lax.fori_loop(0, _NSEG, zacc_body, 0)

        def cube_row(xb, r, accs):
            out = []
            for j in range(nch):
                v = jnp.maximum(xb[r, pl.ds(j * 16, 16)], _EPS)
                out.append(accs[j] + v * v * v)
            return tuple(out)

        for b in range(nblk):
            xbuf = bufs[b % 2]
            pltpu.make_async_copy(x_hbm.at[pl.ds(0, blk)], xbuf,
                                  sems[b % 2]).wait()
            blo = b * blk

            def seg_body(s, carry2, _xbuf=xbuf, _blo=blo):
                st = sbuf[pl.ds(s, 16)][0]
                en = sbuf[pl.ds(16 + s, 16)][0]
                lo = jnp.clip(st, _blo, _blo + blk) - _blo
                hi = jnp.clip(en, _blo, _blo + blk) - _blo

                @pl.when(hi > lo)
                def _():
                    accs = lax.fori_loop(
                        lo, hi, lambda r, a: cube_row(_xbuf, r, a),
                        tuple(zero16 for _ in range(nch)))
                    for j in range(nch):
                        acc[s, pl.ds(j * 16, 16)] += accs[j]

                return carry2

            lax.fori_loop(0, _NSEG, seg_body, 0)

            if b + 2 < nblk:
                pltpu.async_copy(
                    x_hbm.at[pl.ds(base + (b + 2) * blk, blk)],
                    bufs[b % 2], sems[b % 2])
        pltpu.sync_copy(acc, psum_hbm.at[wid])
        pltpu.sync_copy(cntf, pcnt_hbm.at[wid])

    return sc_cube


_TCBLK = 2048


@functools.lru_cache(maxsize=None)
def _make_tc_partial(t0, d, cube):
    nblk = t0 // _TCBLK

    def body(x_ref, ids_ref, p_ref, psum_ref, pcnt_ref):
        @pl.when(pl.program_id(0) == 0)
        def _():
            psum_ref[...] = jnp.zeros_like(psum_ref)
            pcnt_ref[...] = jnp.zeros_like(pcnt_ref)

        ids = ids_ref[0, 0, :]
        smin = jnp.min(ids)
        smax = jnp.max(ids)
        v = jnp.maximum(x_ref[...], _EPS)
        feats = v * v * v if cube else v ** p_ref[0, 0]

        @pl.when(smin == smax)
        def _():
            colsum = jnp.sum(feats, axis=0)
            psum_ref[pl.ds(smin, 1), :] += colsum[None, :]
            pcnt_ref[pl.ds(smin, 1), :] += jnp.float32(_TCBLK)

        @pl.when(smin != smax)
        def _():
            oh = (ids[None, :] == lax.broadcasted_iota(
                jnp.int32, (_NSEG, _TCBLK), 0)).astype(jnp.float32)
            psum_ref[...] += jnp.dot(oh, feats,
                                     preferred_element_type=jnp.float32)
            pcnt_ref[...] += jnp.broadcast_to(
                jnp.sum(oh, axis=1)[:, None], (_NSEG, 128))

    return pl.pallas_call(
        body,
        grid=(nblk,),
        in_specs=[
            pl.BlockSpec((_TCBLK, d), lambda i: (i, 0)),
            pl.BlockSpec((1, 1, _TCBLK), lambda i: (i, 0, 0)),
            pl.BlockSpec((1, 1), lambda i: (0, 0)),
        ],
        out_specs=[
            pl.BlockSpec((_NSEG, d), lambda i: (0, 0)),
            pl.BlockSpec((_NSEG, 128), lambda i: (0, 0)),
        ],
        out_shape=[
            jax.ShapeDtypeStruct((_NSEG, d), jnp.float32),
            jax.ShapeDtypeStruct((_NSEG, 128), jnp.float32),
        ],
    )


def _finish_body(psum_ref, pcnt_ref, tsum_ref, tcnt_ref, p_ref, o_ref):
    p = p_ref[0, 0]
    s = jnp.sum(psum_ref[...], axis=0) + tsum_ref[...]
    c = (jnp.sum(pcnt_ref[...], axis=0) + tcnt_ref[:, 0])[:, None]
    mean = s / jnp.maximum(c, 1.0)
    o_ref[...] = mean ** (1.0 / p)


@functools.lru_cache(maxsize=None)
def _make_finish(d):
    return pl.pallas_call(
        _finish_body,
        out_shape=jax.ShapeDtypeStruct((_NSEG, d), jnp.float32),
    )


_T0 = 16384


def kernel(x, p, batch_ids):
    total, d = x.shape
    bid = batch_ids.astype(jnp.int32)
    bid2d = bid.reshape(total // _TCBLK, 1, _TCBLK)
    pf = p.astype(jnp.float32)
    p11 = pf.reshape(1, 1)
    pln2 = jnp.broadcast_to(pf * _LN2, (16,))

    def cube_branch(xx, bb, bb2, pp):
        ps, pc = _make_sc_cube(total, d, _T0)(xx, bb)
        ts, tc = _make_tc_partial(_T0, d, True)(xx, bb2, pp)
        return ps, pc, ts, tc

    def gen_branch(xx, bb, bb2, pp):
        ps, pc = _make_sc_pool(total, d, _T0)(xx, bb, pln2)
        ts, tc = _make_tc_partial(_T0, d, False)(xx, bb2, pp)
        return ps, pc, ts, tc

    psum, pcnt, tsum, tcnt = lax.cond(
        pf[0] == 3.0, cube_branch, gen_branch, x, bid, bid2d, p11)
    return _make_finish(d)(psum, pcnt, tsum, tcnt, p11)


# speedup vs baseline: 1.0993x; 1.0511x over previous
import functools

import jax
import jax.numpy as jnp
from jax import lax
from jax.experimental import pallas as pl
from jax.experimental.pallas import tpu as pltpu
from jax.experimental.pallas import tpu_sc as plsc

_EPS = 1e-06
_NSEG = 16
_LN2 = 0.6931471805599453
_C1 = 1.4415923923106588
_C2 = -0.7072548989690162
_C3 = 0.4115641479248821
_C4 = -0.18983442828200595
_C5 = 0.04392909981021807

_NW = 32
_BLK = 256


def _pow_p(v, pln2):
    v = jnp.maximum(v, _EPS)
    iv = lax.bitcast_convert_type(v, jnp.int32)
    e = (iv >> 23) - 127
    m = lax.bitcast_convert_type((iv & 0x007FFFFF) | 0x3F800000, jnp.float32)
    t = m - 1.0
    lg = t * (_C1 + t * (_C2 + t * (_C3 + t * (_C4 + t * _C5))))
    lg = lg + e.astype(jnp.float32)
    return jnp.exp(pln2 * lg)


@functools.lru_cache(maxsize=None)
def _make_sc_pool(total, d, t0=0):
    rows = (total - t0) // _NW
    nblk = rows // _BLK
    nch = d // 16
    nidc = rows // 16
    mesh = plsc.VectorSubcoreMesh(core_axis_name="c", subcore_axis_name="s")

    @functools.partial(
        pl.kernel,
        mesh=mesh,
        out_type=[
            jax.ShapeDtypeStruct((_NW, _NSEG, d), jnp.float32),
            jax.ShapeDtypeStruct((_NW, _NSEG), jnp.float32),
        ],
        scratch_types=[
            pltpu.VMEM((_BLK, d), jnp.float32),
            pltpu.VMEM((rows + 16,), jnp.int32),
            pltpu.VMEM((_NSEG, d), jnp.float32),
            pltpu.VMEM((16,), jnp.float32),
            pltpu.VMEM((48,), jnp.int32),
            pltpu.VMEM((16,), jnp.float32),
        ],
    )
    def sc_pool(x_hbm, bid_hbm, pln2_hbm, psum_hbm, pcnt_hbm,
                xbuf, bidv, acc, cntf, sbuf, pv):
        wid = lax.axis_index("s") * 2 + lax.axis_index("c")
        base = t0 + wid * rows
        pltpu.sync_copy(bid_hbm.at[pl.ds(base, rows)], bidv.at[pl.ds(0, rows)])
        pltpu.sync_copy(pln2_hbm, pv)
        pln2 = pv[...]
        zero16 = jnp.zeros((16,), jnp.float32)

        def searchsorted(sval):
            lo = jnp.int32(0)
            hi = jnp.int32(rows)
            for _ in range(11):
                mid = jnp.minimum((lo + hi) >> 1, rows - 1)
                v = bidv[pl.ds(mid, 16)][0]
                upd = hi > lo
                lt = v < sval
                lo = jnp.where(upd & lt, mid + 1, lo)
                hi = jnp.where(upd & (~lt), mid, hi)
            return lo

        bounds = [jnp.int32(0)] + [searchsorted(s) for s in range(1, 16)]
        seg_iota = lax.iota(jnp.int32, 16)
        starts = jnp.zeros((16,), jnp.int32)
        ends = jnp.full((16,), rows, jnp.int32)
        for s in range(1, 16):
            bs = jnp.full((16,), bounds[s], jnp.int32)
            starts = jnp.where(seg_iota == s, bs, starts)
            ends = jnp.where(seg_iota == s - 1, bs, ends)
        sbuf[pl.ds(0, 16)] = starts
        sbuf[pl.ds(16, 16)] = ends
        sbuf[pl.ds(32, 16)] = jnp.zeros((16,), jnp.int32)
        cntf[...] = (ends - starts).astype(jnp.float32)

        def zacc_body(s, carry):
            for j in range(nch):
                acc[s, pl.ds(j * 16, 16)] = zero16
            return carry

        lax.fori_loop(0, _NSEG, zacc_body, 0)

        def block_body(b, carry):
            pltpu.sync_copy(x_hbm.at[pl.ds(base + b * _BLK, _BLK)], xbuf)
            blo = b * _BLK

            def seg_body(s, carry2):
                st = sbuf[pl.ds(s, 16)][0]
                en = sbuf[pl.ds(16 + s, 16)][0]
                lo = jnp.clip(st, blo, blo + _BLK) - blo
                hi = jnp.clip(en, blo, blo + _BLK) - blo

                def row_body(r, accs):
                    return tuple(
                        accs[j] + _pow_p(xbuf[r, pl.ds(j * 16, 16)], pln2)
                        for j in range(nch)
                    )

                accs = lax.fori_loop(lo, hi, row_body,
                                     tuple(zero16 for _ in range(nch)))
                for j in range(nch):
                    acc[s, pl.ds(j * 16, 16)] += accs[j]
                return carry2

            lax.fori_loop(0, _NSEG, seg_body, 0)
            return carry

        lax.fori_loop(0, nblk, block_body, 0)
        pltpu.sync_copy(acc, psum_hbm.at[wid])
        pltpu.sync_copy(cntf, pcnt_hbm.at[wid])

    return sc_pool


@functools.lru_cache(maxsize=None)
def _make_sc_cube(total, d, t0=0):
    rows = (total - t0) // _NW
    blk = 128
    nblk = rows // blk
    nch = d // 16
    mesh = plsc.VectorSubcoreMesh(core_axis_name="c", subcore_axis_name="s")

    @functools.partial(
        pl.kernel,
        mesh=mesh,
        out_type=[
            jax.ShapeDtypeStruct((_NW, _NSEG, d), jnp.float32),
            jax.ShapeDtypeStruct((_NW, _NSEG), jnp.float32),
        ],
        scratch_types=[
            pltpu.VMEM((blk, d), jnp.float32),
            pltpu.VMEM((blk, d), jnp.float32),
            pltpu.VMEM((rows + 16,), jnp.int32),
            pltpu.VMEM((_NSEG, d), jnp.float32),
            pltpu.VMEM((16,), jnp.float32),
            pltpu.VMEM((48,), jnp.int32),
            pltpu.SemaphoreType.DMA,
            pltpu.SemaphoreType.DMA,
        ],
    )
    def sc_cube(x_hbm, bid_hbm, psum_hbm, pcnt_hbm,
                xbuf0, xbuf1, bidv, acc, cntf, sbuf, sem0, sem1):
        wid = lax.axis_index("s") * 2 + lax.axis_index("c")
        base = t0 + wid * rows
        bufs = (xbuf0, xbuf1)
        sems = (sem0, sem1)
        pltpu.async_copy(x_hbm.at[pl.ds(base, blk)], bufs[0], sems[0])
        pltpu.async_copy(x_hbm.at[pl.ds(base + blk, blk)], bufs[1], sems[1])
        pltpu.sync_copy(bid_hbm.at[pl.ds(base, rows)], bidv.at[pl.ds(0, rows)])
        zero16 = jnp.zeros((16,), jnp.float32)

        nsteps = max(1, (rows).bit_length())

        def bs_body(i, lh):
            los, his = lh
            nlo, nhi = [], []
            for k in range(15):
                lo, hi = los[k], his[k]
                mid = jnp.minimum((lo + hi) >> 1, rows - 1)
                v = bidv[pl.ds(mid, 16)][0]
                upd = hi > lo
                lt = v < (k + 1)
                nlo.append(jnp.where(upd & lt, mid + 1, lo))
                nhi.append(jnp.where(upd & (~lt), mid, hi))
            return (tuple(nlo), tuple(nhi))

        los, _ = lax.fori_loop(
            0, nsteps, bs_body,
            (tuple([jnp.int32(0)] * 15), tuple([jnp.int32(rows)] * 15)))
        seg_iota = lax.iota(jnp.int32, 16)
        starts = jnp.zeros((16,), jnp.int32)
        ends = jnp.full((16,), rows, jnp.int32)
        for s in range(1, 16):
            bs = jnp.full((16,), los[s - 1], jnp.int32)
            starts = jnp.where(seg_iota == s, bs, starts)
            ends = jnp.where(seg_iota == s - 1, bs, ends)
        sbuf[pl.ds(0, 16)] = starts
        sbuf[pl.ds(16, 16)] = ends
        sbuf[pl.ds(32, 16)] = jnp.zeros((16,), jnp.int32)
        cntf[...] = (ends - starts).astype(jnp.float32)

        def zacc_body(s, carry):
            for j in range(nch):
                acc[s, pl.ds(j * 16, 16)] = zero16
            return carry

        lax.fori_loop(0, _NSEG, zacc_body, 0)

        def cube_row(xb, r, accs):
            out = []
            for j in range(nch):
                v = jnp.maximum(xb[r, pl.ds(j * 16, 16)], _EPS)
                out.append(accs[j] + v * v * v)
            return tuple(out)

        def pair_body(g, carry):
            for u in range(2):
                b = g * 2 + u
                xbuf = bufs[u]
                sem = sems[u]
                pltpu.make_async_copy(
                    x_hbm.at[pl.ds(0, blk)], xbuf, sem).wait()
                blo = b * blk

                def seg_body(s, carry2, _xbuf=xbuf, _blo=blo):
                    st = sbuf[pl.ds(s, 16)][0]
                    en = sbuf[pl.ds(16 + s, 16)][0]
                    lo = jnp.clip(st, _blo, _blo + blk) - _blo
                    hi = jnp.clip(en, _blo, _blo + blk) - _blo

                    @pl.when(hi > lo)
                    def _():
                        accs = lax.fori_loop(
                            lo, hi, lambda r, a: cube_row(_xbuf, r, a),
                            tuple(zero16 for _ in range(nch)))
                        for j in range(nch):
                            acc[s, pl.ds(j * 16, 16)] += accs[j]

                    return carry2

                lax.fori_loop(0, _NSEG, seg_body, 0)

                @pl.when(b + 2 < nblk)
                def _():
                    pltpu.async_copy(
                        x_hbm.at[pl.ds(base + (b + 2) * blk, blk)], xbuf, sem)

            return carry

        lax.fori_loop(0, nblk // 2, pair_body, 0)
        pltpu.sync_copy(acc, psum_hbm.at[wid])
        pltpu.sync_copy(cntf, pcnt_hbm.at[wid])

    return sc_cube


_TCBLK = 2048


@functools.lru_cache(maxsize=None)
def _make_tc_partial(t0, d, cube):
    nblk = t0 // _TCBLK

    def body(x_ref, ids_ref, p_ref, psum_ref, pcnt_ref):
        @pl.when(pl.program_id(0) == 0)
        def _():
            psum_ref[...] = jnp.zeros_like(psum_ref)
            pcnt_ref[...] = jnp.zeros_like(pcnt_ref)

        ids = ids_ref[0, 0, :]
        smin = jnp.min(ids)
        smax = jnp.max(ids)
        v = jnp.maximum(x_ref[...], _EPS)
        feats = v * v * v if cube else v ** p_ref[0, 0]

        @pl.when(smin == smax)
        def _():
            colsum = jnp.sum(feats, axis=0)
            psum_ref[pl.ds(smin, 1), :] += colsum[None, :]
            pcnt_ref[pl.ds(smin, 1), :] += jnp.float32(_TCBLK)

        @pl.when(smin != smax)
        def _():
            oh = (ids[None, :] == lax.broadcasted_iota(
                jnp.int32, (_NSEG, _TCBLK), 0)).astype(jnp.float32)
            psum_ref[...] += jnp.dot(oh, feats,
                                     preferred_element_type=jnp.float32)
            pcnt_ref[...] += jnp.broadcast_to(
                jnp.sum(oh, axis=1)[:, None], (_NSEG, 128))

    return pl.pallas_call(
        body,
        grid=(nblk,),
        in_specs=[
            pl.BlockSpec((_TCBLK, d), lambda i: (i, 0)),
            pl.BlockSpec((1, 1, _TCBLK), lambda i: (i, 0, 0)),
            pl.BlockSpec((1, 1), lambda i: (0, 0)),
        ],
        out_specs=[
            pl.BlockSpec((_NSEG, d), lambda i: (0, 0)),
            pl.BlockSpec((_NSEG, 128), lambda i: (0, 0)),
        ],
        out_shape=[
            jax.ShapeDtypeStruct((_NSEG, d), jnp.float32),
            jax.ShapeDtypeStruct((_NSEG, 128), jnp.float32),
        ],
    )


def _finish_body(psum_ref, pcnt_ref, tsum_ref, tcnt_ref, p_ref, o_ref):
    p = p_ref[0, 0]
    s = jnp.sum(psum_ref[...], axis=0) + tsum_ref[...]
    c = (jnp.sum(pcnt_ref[...], axis=0) + tcnt_ref[:, 0])[:, None]
    mean = s / jnp.maximum(c, 1.0)
    o_ref[...] = mean ** (1.0 / p)


@functools.lru_cache(maxsize=None)
def _make_finish(d):
    return pl.pallas_call(
        _finish_body,
        out_shape=jax.ShapeDtypeStruct((_NSEG, d), jnp.float32),
    )


_T0 = 16384


def kernel(x, p, batch_ids):
    total, d = x.shape
    bid = batch_ids.astype(jnp.int32)
    bid2d = bid.reshape(total // _TCBLK, 1, _TCBLK)
    pf = p.astype(jnp.float32)
    p11 = pf.reshape(1, 1)
    pln2 = jnp.broadcast_to(pf * _LN2, (16,))

    def cube_branch(xx, bb, bb2, pp):
        ps, pc = _make_sc_cube(total, d, _T0)(xx, bb)
        ts, tc = _make_tc_partial(_T0, d, True)(xx, bb2, pp)
        return ps, pc, ts, tc

    def gen_branch(xx, bb, bb2, pp):
        ps, pc = _make_sc_pool(total, d, _T0)(xx, bb, pln2)
        ts, tc = _make_tc_partial(_T0, d, False)(xx, bb2, pp)
        return ps, pc, ts, tc

    psum, pcnt, tsum, tcnt = lax.cond(
        pf[0] == 3.0, cube_branch, gen_branch, x, bid, bid2d, p11)
    return _make_finish(d)(psum, pcnt, tsum, tcnt, p11)


# speedup vs baseline: 1.1206x; 1.0194x over previous
import functools

import jax
import jax.numpy as jnp
from jax import lax
from jax.experimental import pallas as pl
from jax.experimental.pallas import tpu as pltpu
from jax.experimental.pallas import tpu_sc as plsc

_EPS = 1e-06
_NSEG = 16
_LN2 = 0.6931471805599453
_C1 = 1.4415923923106588
_C2 = -0.7072548989690162
_C3 = 0.4115641479248821
_C4 = -0.18983442828200595
_C5 = 0.04392909981021807

_NW = 32
_BLK = 256


def _pow_p(v, pln2):
    v = jnp.maximum(v, _EPS)
    iv = lax.bitcast_convert_type(v, jnp.int32)
    e = (iv >> 23) - 127
    m = lax.bitcast_convert_type((iv & 0x007FFFFF) | 0x3F800000, jnp.float32)
    t = m - 1.0
    lg = t * (_C1 + t * (_C2 + t * (_C3 + t * (_C4 + t * _C5))))
    lg = lg + e.astype(jnp.float32)
    return jnp.exp(pln2 * lg)


@functools.lru_cache(maxsize=None)
def _make_sc_pool(total, d, t0=0):
    rows = (total - t0) // _NW
    nblk = rows // _BLK
    nch = d // 16
    nidc = rows // 16
    mesh = plsc.VectorSubcoreMesh(core_axis_name="c", subcore_axis_name="s")

    @functools.partial(
        pl.kernel,
        mesh=mesh,
        out_type=[
            jax.ShapeDtypeStruct((_NW, _NSEG, d), jnp.float32),
            jax.ShapeDtypeStruct((_NW, _NSEG), jnp.float32),
        ],
        scratch_types=[
            pltpu.VMEM((_BLK, d), jnp.float32),
            pltpu.VMEM((rows + 16,), jnp.int32),
            pltpu.VMEM((_NSEG, d), jnp.float32),
            pltpu.VMEM((16,), jnp.float32),
            pltpu.VMEM((48,), jnp.int32),
            pltpu.VMEM((16,), jnp.float32),
        ],
    )
    def sc_pool(x_hbm, bid_hbm, pln2_hbm, psum_hbm, pcnt_hbm,
                xbuf, bidv, acc, cntf, sbuf, pv):
        wid = lax.axis_index("s") * 2 + lax.axis_index("c")
        base = t0 + wid * rows
        pltpu.sync_copy(bid_hbm.at[pl.ds(base, rows)], bidv.at[pl.ds(0, rows)])
        pltpu.sync_copy(pln2_hbm, pv)
        pln2 = pv[...]
        zero16 = jnp.zeros((16,), jnp.float32)

        def searchsorted(sval):
            def bs(i, lh):
                lo, hi = lh
                mid = jnp.minimum((lo + hi) >> 1, rows - 1)
                v = bidv[pl.ds(mid, 16)][0]
                upd = hi > lo
                lt = v < sval
                return (jnp.where(upd & lt, mid + 1, lo),
                        jnp.where(upd & (~lt), mid, hi))

            lo, _ = lax.fori_loop(0, max(1, rows.bit_length()), bs,
                                  (jnp.int32(0), jnp.int32(rows)))
            return lo

        bounds = [jnp.int32(0)] + [searchsorted(s) for s in range(1, 16)]
        seg_iota = lax.iota(jnp.int32, 16)
        starts = jnp.zeros((16,), jnp.int32)
        ends = jnp.full((16,), rows, jnp.int32)
        for s in range(1, 16):
            bs = jnp.full((16,), bounds[s], jnp.int32)
            starts = jnp.where(seg_iota == s, bs, starts)
            ends = jnp.where(seg_iota == s - 1, bs, ends)
        sbuf[pl.ds(0, 16)] = starts
        sbuf[pl.ds(16, 16)] = ends
        sbuf[pl.ds(32, 16)] = jnp.zeros((16,), jnp.int32)
        cntf[...] = (ends - starts).astype(jnp.float32)

        def zacc_body(s, carry):
            for j in range(nch):
                acc[s, pl.ds(j * 16, 16)] = zero16
            return carry

        lax.fori_loop(0, _NSEG, zacc_body, 0)

        def block_body(b, carry):
            pltpu.sync_copy(x_hbm.at[pl.ds(base + b * _BLK, _BLK)], xbuf)
            blo = b * _BLK

            def seg_body(s, carry2):
                st = sbuf[pl.ds(s, 16)][0]
                en = sbuf[pl.ds(16 + s, 16)][0]
                lo = jnp.clip(st, blo, blo + _BLK) - blo
                hi = jnp.clip(en, blo, blo + _BLK) - blo

                def row_body(r, accs):
                    return tuple(
                        accs[j] + _pow_p(xbuf[r, pl.ds(j * 16, 16)], pln2)
                        for j in range(nch)
                    )

                accs = lax.fori_loop(lo, hi, row_body,
                                     tuple(zero16 for _ in range(nch)))
                for j in range(nch):
                    acc[s, pl.ds(j * 16, 16)] += accs[j]
                return carry2

            lax.fori_loop(0, _NSEG, seg_body, 0)
            return carry

        lax.fori_loop(0, nblk, block_body, 0)
        pltpu.sync_copy(acc, psum_hbm.at[wid])
        pltpu.sync_copy(cntf, pcnt_hbm.at[wid])

    return sc_pool


@functools.lru_cache(maxsize=None)
def _make_sc_cube(total, d, t0=0):
    rows = (total - t0) // _NW
    nblk = 4
    blk = rows // nblk
    nch = d // 16
    mesh = plsc.VectorSubcoreMesh(core_axis_name="c", subcore_axis_name="s")

    @functools.partial(
        pl.kernel,
        mesh=mesh,
        out_type=[
            jax.ShapeDtypeStruct((_NW, _NSEG, d), jnp.float32),
            jax.ShapeDtypeStruct((_NW, _NSEG), jnp.float32),
        ],
        scratch_types=[
            pltpu.VMEM((blk, d), jnp.float32),
            pltpu.VMEM((blk, d), jnp.float32),
            pltpu.VMEM((rows + 16,), jnp.int32),
            pltpu.VMEM((_NSEG, d), jnp.float32),
            pltpu.VMEM((16,), jnp.float32),
            pltpu.VMEM((48,), jnp.int32),
            pltpu.SemaphoreType.DMA,
            pltpu.SemaphoreType.DMA,
        ],
    )
    def sc_cube(x_hbm, bid_hbm, psum_hbm, pcnt_hbm,
                xbuf0, xbuf1, bidv, acc, cntf, sbuf, sem0, sem1):
        wid = lax.axis_index("s") * 2 + lax.axis_index("c")
        base = t0 + wid * rows
        bufs = (xbuf0, xbuf1)
        sems = (sem0, sem1)
        pltpu.async_copy(x_hbm.at[pl.ds(base, blk)], bufs[0], sems[0])
        pltpu.async_copy(x_hbm.at[pl.ds(base + blk, blk)], bufs[1], sems[1])
        pltpu.sync_copy(bid_hbm.at[pl.ds(base, rows)], bidv.at[pl.ds(0, rows)])
        zero16 = jnp.zeros((16,), jnp.float32)

        nsteps = max(1, (rows).bit_length())

        def bs_body(i, lh):
            los, his = lh
            nlo, nhi = [], []
            for k in range(15):
                lo, hi = los[k], his[k]
                mid = jnp.minimum((lo + hi) >> 1, rows - 1)
                v = bidv[pl.ds(mid, 16)][0]
                upd = hi > lo
                lt = v < (k + 1)
                nlo.append(jnp.where(upd & lt, mid + 1, lo))
                nhi.append(jnp.where(upd & (~lt), mid, hi))
            return (tuple(nlo), tuple(nhi))

        los, _ = lax.fori_loop(
            0, nsteps, bs_body,
            (tuple([jnp.int32(0)] * 15), tuple([jnp.int32(rows)] * 15)))
        seg_iota = lax.iota(jnp.int32, 16)
        starts = jnp.zeros((16,), jnp.int32)
        ends = jnp.full((16,), rows, jnp.int32)
        for s in range(1, 16):
            bs = jnp.full((16,), los[s - 1], jnp.int32)
            starts = jnp.where(seg_iota == s, bs, starts)
            ends = jnp.where(seg_iota == s - 1, bs, ends)
        sbuf[pl.ds(0, 16)] = starts
        sbuf[pl.ds(16, 16)] = ends
        sbuf[pl.ds(32, 16)] = jnp.zeros((16,), jnp.int32)
        cntf[...] = (ends - starts).astype(jnp.float32)

        def zacc_body(s, carry):
            for j in range(nch):
                acc[s, pl.ds(j * 16, 16)] = zero16
            return carry

        lax.fori_loop(0, _NSEG, zacc_body, 0)

        def cube_row(xb, r, accs):
            out = []
            for j in range(nch):
                v = jnp.maximum(xb[r, pl.ds(j * 16, 16)], _EPS)
                out.append(accs[j] + v * v * v)
            return tuple(out)

        def pair_body(g, carry):
            for u in range(2):
                b = g * 2 + u
                xbuf = bufs[u]
                sem = sems[u]
                pltpu.make_async_copy(
                    x_hbm.at[pl.ds(0, blk)], xbuf, sem).wait()
                blo = b * blk

                def seg_body(s, carry2, _xbuf=xbuf, _blo=blo):
                    st = sbuf[pl.ds(s, 16)][0]
                    en = sbuf[pl.ds(16 + s, 16)][0]
                    lo = jnp.clip(st, _blo, _blo + blk) - _blo
                    hi = jnp.clip(en, _blo, _blo + blk) - _blo

                    @pl.when(hi > lo)
                    def _():
                        accs = lax.fori_loop(
                            lo, hi, lambda r, a: cube_row(_xbuf, r, a),
                            tuple(zero16 for _ in range(nch)))
                        for j in range(nch):
                            acc[s, pl.ds(j * 16, 16)] += accs[j]

                    return carry2

                lax.fori_loop(0, _NSEG, seg_body, 0)

                @pl.when(b + 2 < nblk)
                def _():
                    pltpu.async_copy(
                        x_hbm.at[pl.ds(base + (b + 2) * blk, blk)], xbuf, sem)

            return carry

        lax.fori_loop(0, nblk // 2, pair_body, 0)
        pltpu.sync_copy(acc, psum_hbm.at[wid])
        pltpu.sync_copy(cntf, pcnt_hbm.at[wid])

    return sc_cube


_TCBLK = 2048


@functools.lru_cache(maxsize=None)
def _make_tc_partial(t0, d, cube):
    nblk = t0 // _TCBLK

    def body(x_ref, ids_ref, p_ref, psum_ref, pcnt_ref):
        @pl.when(pl.program_id(0) == 0)
        def _():
            psum_ref[...] = jnp.zeros_like(psum_ref)
            pcnt_ref[...] = jnp.zeros_like(pcnt_ref)

        ids = ids_ref[0, 0, :]
        smin = jnp.min(ids)
        smax = jnp.max(ids)
        v = jnp.maximum(x_ref[...], _EPS)
        feats = v * v * v if cube else v ** p_ref[0, 0]

        @pl.when(smin == smax)
        def _():
            colsum = jnp.sum(feats, axis=0)
            psum_ref[pl.ds(smin, 1), :] += colsum[None, :]
            pcnt_ref[pl.ds(smin, 1), :] += jnp.float32(_TCBLK)

        @pl.when(smin != smax)
        def _():
            oh = (ids[None, :] == lax.broadcasted_iota(
                jnp.int32, (_NSEG, _TCBLK), 0)).astype(jnp.float32)
            psum_ref[...] += jnp.dot(oh, feats,
                                     preferred_element_type=jnp.float32)
            pcnt_ref[...] += jnp.broadcast_to(
                jnp.sum(oh, axis=1)[:, None], (_NSEG, 128))

    return pl.pallas_call(
        body,
        grid=(nblk,),
        in_specs=[
            pl.BlockSpec((_TCBLK, d), lambda i: (i, 0)),
            pl.BlockSpec((1, 1, _TCBLK), lambda i: (i, 0, 0)),
            pl.BlockSpec((1, 1), lambda i: (0, 0)),
        ],
        out_specs=[
            pl.BlockSpec((_NSEG, d), lambda i: (0, 0)),
            pl.BlockSpec((_NSEG, 128), lambda i: (0, 0)),
        ],
        out_shape=[
            jax.ShapeDtypeStruct((_NSEG, d), jnp.float32),
            jax.ShapeDtypeStruct((_NSEG, 128), jnp.float32),
        ],
    )


def _finish_body(psum_ref, pcnt_ref, tsum_ref, tcnt_ref, p_ref, o_ref):
    p = p_ref[0, 0]
    s = jnp.sum(psum_ref[...], axis=0) + tsum_ref[...]
    c = (jnp.sum(pcnt_ref[...], axis=0) + tcnt_ref[:, 0])[:, None]
    mean = s / jnp.maximum(c, 1.0)
    o_ref[...] = mean ** (1.0 / p)


@functools.lru_cache(maxsize=None)
def _make_finish(d):
    return pl.pallas_call(
        _finish_body,
        out_shape=jax.ShapeDtypeStruct((_NSEG, d), jnp.float32),
    )


_T0 = 18432


def kernel(x, p, batch_ids):
    total, d = x.shape
    bid = batch_ids.astype(jnp.int32)
    bid2d = bid.reshape(total // _TCBLK, 1, _TCBLK)
    pf = p.astype(jnp.float32)
    p11 = pf.reshape(1, 1)
    pln2 = jnp.broadcast_to(pf * _LN2, (16,))

    def cube_branch(xx, bb, bb2, pp):
        ps, pc = _make_sc_cube(total, d, _T0)(xx, bb)
        ts, tc = _make_tc_partial(_T0, d, True)(xx, bb2, pp)
        return ps, pc, ts, tc

    def gen_branch(xx, bb, bb2, pp):
        ps, pc = _make_sc_pool(total, d, _T0)(xx, bb, pln2)
        ts, tc = _make_tc_partial(_T0, d, False)(xx, bb2, pp)
        return ps, pc, ts, tc

    psum, pcnt, tsum, tcnt = lax.cond(
        pf[0] == 3.0, cube_branch, gen_branch, x, bid, bid2d, p11)
    return _make_finish(d)(psum, pcnt, tsum, tcnt, p11)


# speedup vs baseline: 1.1562x; 1.0318x over previous
import functools

import jax
import jax.numpy as jnp
from jax import lax
from jax.experimental import pallas as pl
from jax.experimental.pallas import tpu as pltpu
from jax.experimental.pallas import tpu_sc as plsc

_EPS = 1e-06
_NSEG = 16
_LN2 = 0.6931471805599453
_C1 = 1.4415923923106588
_C2 = -0.7072548989690162
_C3 = 0.4115641479248821
_C4 = -0.18983442828200595
_C5 = 0.04392909981021807

_NW = 32
_BLK = 256


def _pow_p(v, pln2):
    v = jnp.maximum(v, _EPS)
    iv = lax.bitcast_convert_type(v, jnp.int32)
    e = (iv >> 23) - 127
    m = lax.bitcast_convert_type((iv & 0x007FFFFF) | 0x3F800000, jnp.float32)
    t = m - 1.0
    lg = t * (_C1 + t * (_C2 + t * (_C3 + t * (_C4 + t * _C5))))
    lg = lg + e.astype(jnp.float32)
    return jnp.exp(pln2 * lg)


@functools.lru_cache(maxsize=None)
def _make_sc_pool(total, d, t0=0):
    rows = (total - t0) // _NW
    nblk = rows // _BLK
    nch = d // 16
    nidc = rows // 16
    mesh = plsc.VectorSubcoreMesh(core_axis_name="c", subcore_axis_name="s")

    @functools.partial(
        pl.kernel,
        mesh=mesh,
        out_type=[
            jax.ShapeDtypeStruct((_NW, _NSEG, d), jnp.float32),
            jax.ShapeDtypeStruct((_NW, _NSEG), jnp.float32),
        ],
        scratch_types=[
            pltpu.VMEM((_BLK, d), jnp.float32),
            pltpu.VMEM((rows + 16,), jnp.int32),
            pltpu.VMEM((_NSEG, d), jnp.float32),
            pltpu.VMEM((16,), jnp.float32),
            pltpu.VMEM((48,), jnp.int32),
            pltpu.VMEM((16,), jnp.float32),
        ],
    )
    def sc_pool(x_hbm, bid_hbm, pln2_hbm, psum_hbm, pcnt_hbm,
                xbuf, bidv, acc, cntf, sbuf, pv):
        wid = lax.axis_index("s") * 2 + lax.axis_index("c")
        base = t0 + wid * rows
        pltpu.sync_copy(bid_hbm.at[pl.ds(base, rows)], bidv.at[pl.ds(0, rows)])
        pltpu.sync_copy(pln2_hbm, pv)
        pln2 = pv[...]
        zero16 = jnp.zeros((16,), jnp.float32)

        def searchsorted(sval):
            def bs(i, lh):
                lo, hi = lh
                mid = jnp.minimum((lo + hi) >> 1, rows - 1)
                v = bidv[pl.ds(mid, 16)][0]
                upd = hi > lo
                lt = v < sval
                return (jnp.where(upd & lt, mid + 1, lo),
                        jnp.where(upd & (~lt), mid, hi))

            lo, _ = lax.fori_loop(0, max(1, rows.bit_length()), bs,
                                  (jnp.int32(0), jnp.int32(rows)))
            return lo

        bounds = [jnp.int32(0)] + [searchsorted(s) for s in range(1, 16)]
        seg_iota = lax.iota(jnp.int32, 16)
        starts = jnp.zeros((16,), jnp.int32)
        ends = jnp.full((16,), rows, jnp.int32)
        for s in range(1, 16):
            bs = jnp.full((16,), bounds[s], jnp.int32)
            starts = jnp.where(seg_iota == s, bs, starts)
            ends = jnp.where(seg_iota == s - 1, bs, ends)
        sbuf[pl.ds(0, 16)] = starts
        sbuf[pl.ds(16, 16)] = ends
        sbuf[pl.ds(32, 16)] = jnp.zeros((16,), jnp.int32)
        cntf[...] = (ends - starts).astype(jnp.float32)

        def zacc_body(s, carry):
            for j in range(nch):
                acc[s, pl.ds(j * 16, 16)] = zero16
            return carry

        lax.fori_loop(0, _NSEG, zacc_body, 0)

        def block_body(b, carry):
            pltpu.sync_copy(x_hbm.at[pl.ds(base + b * _BLK, _BLK)], xbuf)
            blo = b * _BLK

            def seg_body(s, carry2):
                st = sbuf[pl.ds(s, 16)][0]
                en = sbuf[pl.ds(16 + s, 16)][0]
                lo = jnp.clip(st, blo, blo + _BLK) - blo
                hi = jnp.clip(en, blo, blo + _BLK) - blo

                def row_body(r, accs):
                    return tuple(
                        accs[j] + _pow_p(xbuf[r, pl.ds(j * 16, 16)], pln2)
                        for j in range(nch)
                    )

                accs = lax.fori_loop(lo, hi, row_body,
                                     tuple(zero16 for _ in range(nch)))
                for j in range(nch):
                    acc[s, pl.ds(j * 16, 16)] += accs[j]
                return carry2

            lax.fori_loop(0, _NSEG, seg_body, 0)
            return carry

        lax.fori_loop(0, nblk, block_body, 0)
        pltpu.sync_copy(acc, psum_hbm.at[wid])
        pltpu.sync_copy(cntf, pcnt_hbm.at[wid])

    return sc_pool


@functools.lru_cache(maxsize=None)
def _make_sc_cube(total, d, t0=0):
    rows = (total - t0) // _NW
    nblk = 4
    blk = rows // nblk
    nch = d // 16
    mesh = plsc.VectorSubcoreMesh(core_axis_name="c", subcore_axis_name="s")

    @functools.partial(
        pl.kernel,
        mesh=mesh,
        out_type=[
            jax.ShapeDtypeStruct((_NW, _NSEG, d), jnp.float32),
            jax.ShapeDtypeStruct((_NW, _NSEG), jnp.float32),
        ],
        scratch_types=[
            pltpu.VMEM((blk, d), jnp.float32),
            pltpu.VMEM((blk, d), jnp.float32),
            pltpu.VMEM((rows + 16,), jnp.int32),
            pltpu.VMEM((_NSEG, d), jnp.float32),
            pltpu.VMEM((16,), jnp.float32),
            pltpu.VMEM((48,), jnp.int32),
            pltpu.SemaphoreType.DMA,
            pltpu.SemaphoreType.DMA,
        ],
    )
    def sc_cube(x_hbm, bid_hbm, psum_hbm, pcnt_hbm,
                xbuf0, xbuf1, bidv, acc, cntf, sbuf, sem0, sem1):
        wid = lax.axis_index("s") * 2 + lax.axis_index("c")
        base = t0 + wid * rows
        bufs = (xbuf0, xbuf1)
        sems = (sem0, sem1)
        pltpu.async_copy(x_hbm.at[pl.ds(base, blk)], bufs[0], sems[0])
        pltpu.async_copy(x_hbm.at[pl.ds(base + blk, blk)], bufs[1], sems[1])
        pltpu.sync_copy(bid_hbm.at[pl.ds(base, rows)], bidv.at[pl.ds(0, rows)])
        zero16 = jnp.zeros((16,), jnp.float32)

        nsteps = max(1, (rows).bit_length())

        def bs_body(i, lh):
            los, his = lh
            nlo, nhi = [], []
            for k in range(15):
                lo, hi = los[k], his[k]
                mid = jnp.minimum((lo + hi) >> 1, rows - 1)
                v = bidv[pl.ds(mid, 16)][0]
                upd = hi > lo
                lt = v < (k + 1)
                nlo.append(jnp.where(upd & lt, mid + 1, lo))
                nhi.append(jnp.where(upd & (~lt), mid, hi))
            return (tuple(nlo), tuple(nhi))

        los, _ = lax.fori_loop(
            0, nsteps, bs_body,
            (tuple([jnp.int32(0)] * 15), tuple([jnp.int32(rows)] * 15)))
        seg_iota = lax.iota(jnp.int32, 16)
        starts = jnp.zeros((16,), jnp.int32)
        ends = jnp.full((16,), rows, jnp.int32)
        for s in range(1, 16):
            bs = jnp.full((16,), los[s - 1], jnp.int32)
            starts = jnp.where(seg_iota == s, bs, starts)
            ends = jnp.where(seg_iota == s - 1, bs, ends)
        sbuf[pl.ds(0, 16)] = starts
        sbuf[pl.ds(16, 16)] = ends
        sbuf[pl.ds(32, 16)] = jnp.zeros((16,), jnp.int32)
        cntf[...] = (ends - starts).astype(jnp.float32)

        def zacc_body(s, carry):
            for j in range(nch):
                acc[s, pl.ds(j * 16, 16)] = zero16
            return carry

        lax.fori_loop(0, _NSEG, zacc_body, 0)

        def cube_row(xb, r, accs):
            out = []
            for j in range(nch):
                v = jnp.maximum(xb[r, pl.ds(j * 16, 16)], _EPS)
                out.append(accs[j] + v * v * v)
            return tuple(out)

        def pair_body(g, carry):
            for u in range(2):
                b = g * 2 + u
                xbuf = bufs[u]
                sem = sems[u]
                pltpu.make_async_copy(
                    x_hbm.at[pl.ds(0, blk)], xbuf, sem).wait()
                blo = b * blk

                def seg_body(s, carry2, _xbuf=xbuf, _blo=blo):
                    st = sbuf[pl.ds(s, 16)][0]
                    en = sbuf[pl.ds(16 + s, 16)][0]
                    lo = jnp.clip(st, _blo, _blo + blk) - _blo
                    hi = jnp.clip(en, _blo, _blo + blk) - _blo

                    @pl.when(hi > lo)
                    def _():
                        accs = lax.fori_loop(
                            lo, hi, lambda r, a: cube_row(_xbuf, r, a),
                            tuple(zero16 for _ in range(nch)))
                        for j in range(nch):
                            acc[s, pl.ds(j * 16, 16)] += accs[j]

                    return carry2

                lax.fori_loop(0, _NSEG, seg_body, 0)

                @pl.when(b + 2 < nblk)
                def _():
                    pltpu.async_copy(
                        x_hbm.at[pl.ds(base + (b + 2) * blk, blk)], xbuf, sem)

            return carry

        lax.fori_loop(0, nblk // 2, pair_body, 0)
        pltpu.sync_copy(acc, psum_hbm.at[wid])
        pltpu.sync_copy(cntf, pcnt_hbm.at[wid])

    return sc_cube


_TCBLK = 2048


@functools.lru_cache(maxsize=None)
def _make_tc_partial(t0, d, cube):
    nblk = t0 // _TCBLK

    def body(x_ref, ids_ref, p_ref, psum_ref, pcnt_ref):
        @pl.when(pl.program_id(0) == 0)
        def _():
            psum_ref[...] = jnp.zeros_like(psum_ref)
            pcnt_ref[...] = jnp.zeros_like(pcnt_ref)

        ids = ids_ref[0, 0, :]
        smin = jnp.min(ids)
        smax = jnp.max(ids)
        v = jnp.maximum(x_ref[...], _EPS)
        feats = v * v * v if cube else v ** p_ref[0, 0]

        @pl.when(smin == smax)
        def _():
            colsum = jnp.sum(feats, axis=0)
            psum_ref[pl.ds(smin, 1), :] += colsum[None, :]
            pcnt_ref[pl.ds(smin, 1), :] += jnp.float32(_TCBLK)

        @pl.when(smin != smax)
        def _():
            oh = (ids[None, :] == lax.broadcasted_iota(
                jnp.int32, (_NSEG, _TCBLK), 0)).astype(jnp.float32)
            psum_ref[...] += jnp.dot(oh, feats,
                                     preferred_element_type=jnp.float32)
            pcnt_ref[...] += jnp.broadcast_to(
                jnp.sum(oh, axis=1)[:, None], (_NSEG, 128))

    return pl.pallas_call(
        body,
        grid=(nblk,),
        in_specs=[
            pl.BlockSpec((_TCBLK, d), lambda i: (i, 0)),
            pl.BlockSpec((1, 1, _TCBLK), lambda i: (i, 0, 0)),
            pl.BlockSpec((1, 1), lambda i: (0, 0)),
        ],
        out_specs=[
            pl.BlockSpec((_NSEG, d), lambda i: (0, 0)),
            pl.BlockSpec((_NSEG, 128), lambda i: (0, 0)),
        ],
        out_shape=[
            jax.ShapeDtypeStruct((_NSEG, d), jnp.float32),
            jax.ShapeDtypeStruct((_NSEG, 128), jnp.float32),
        ],
    )


def _finish_body(psum_ref, pcnt_ref, tsum_ref, tcnt_ref, p_ref, o_ref):
    p = p_ref[0, 0]
    s = jnp.sum(psum_ref[...], axis=0) + tsum_ref[...]
    c = (jnp.sum(pcnt_ref[...], axis=0) + tcnt_ref[:, 0])[:, None]
    mean = s / jnp.maximum(c, 1.0)
    o_ref[...] = mean ** (1.0 / p)


@functools.lru_cache(maxsize=None)
def _make_finish(d):
    return pl.pallas_call(
        _finish_body,
        out_shape=jax.ShapeDtypeStruct((_NSEG, d), jnp.float32),
    )


_T0 = 20480


def kernel(x, p, batch_ids):
    total, d = x.shape
    bid = batch_ids.astype(jnp.int32)
    bid2d = bid.reshape(total // _TCBLK, 1, _TCBLK)
    pf = p.astype(jnp.float32)
    p11 = pf.reshape(1, 1)
    pln2 = jnp.broadcast_to(pf * _LN2, (16,))

    def cube_branch(xx, bb, bb2, pp):
        ps, pc = _make_sc_cube(total, d, _T0)(xx, bb)
        ts, tc = _make_tc_partial(_T0, d, True)(xx, bb2, pp)
        return ps, pc, ts, tc

    def gen_branch(xx, bb, bb2, pp):
        ps, pc = _make_sc_pool(total, d, _T0)(xx, bb, pln2)
        ts, tc = _make_tc_partial(_T0, d, False)(xx, bb2, pp)
        return ps, pc, ts, tc

    psum, pcnt, tsum, tcnt = lax.cond(
        pf[0] == 3.0, cube_branch, gen_branch, x, bid, bid2d, p11)
    return _make_finish(d)(psum, pcnt, tsum, tcnt, p11)
